# baseline (speedup 1.0000x reference)
import functools

import jax
import jax.numpy as jnp
from jax import lax
from jax.experimental import pallas as pl
from jax.experimental.pallas import tpu as pltpu
from jax.experimental.pallas import tpu_sc as plsc

_LEVELS = [10000, 2500, 640, 160, 64]
_H = 32
_KS = 15
_G = 32
_EPS = 1e-5
_NW = 32
_CHUNK_BYTES = 160 * 1024

_BLOCKS = [
    ('e1_1', 'conv', 1, 64, 1, 0, 0, 'n', 0),
    ('e1_2', 'res', 64, 128, 1, 0, 0, 'n', 0),
    ('e2_1', 'res', 128, 128, 1, 1, 0, 's', 0),
    ('e2_2', 'res', 128, 256, 2, 1, 1, 'n', 1),
    ('e2_3', 'res', 256, 256, 2, 1, 1, 'n', 1),
    ('e3_1', 'res', 256, 256, 2, 2, 1, 's', 1),
    ('e3_2', 'res', 256, 512, 4, 2, 2, 'n', 2),
    ('e3_3', 'res', 512, 512, 4, 2, 2, 'n', 2),
    ('e4_1', 'res', 512, 512, 4, 3, 2, 's', 2),
    ('e4_2', 'res', 512, 1024, 8, 3, 3, 'n', 3),
    ('e4_3', 'res', 1024, 1024, 8, 3, 3, 'n', 3),
    ('e5_1', 'res', 1024, 1024, 8, 4, 3, 's', 3),
    ('e5_2', 'res', 1024, 2048, 16, 4, 4, 'n', 4),
    ('e5_3', 'res', 2048, 2048, 16, 4, 4, 'n', 4),
]


def _rup(x, m):
    return (x + m - 1) // m * m


@functools.lru_cache(maxsize=None)
def _gather_kernel(R, D, nch, CH):
    B = _NW * nch * CH
    mesh = plsc.VectorSubcoreMesh(core_axis_name="c", subcore_axis_name="s")

    @functools.partial(
        pl.kernel,
        out_type=jax.ShapeDtypeStruct((B, D), jnp.float32),
        mesh=mesh,
        scratch_types=[
            pltpu.VMEM((CH,), jnp.int32),
            pltpu.VMEM((CH, D), jnp.float32),
            pltpu.SemaphoreType.DMA,
        ],
        compiler_params=pltpu.CompilerParams(use_tc_tiling_on_sc=False),
    )
    def gk(table_hbm, idx_hbm, out_hbm, idx_v, rows_v, sem):
        wid = lax.axis_index("s") * 2 + lax.axis_index("c")

        def body(i, carry):
            base = (wid * nch + i) * CH
            pltpu.sync_copy(idx_hbm.at[pl.ds(base, CH)], idx_v)
            pltpu.async_copy(table_hbm.at[idx_v], rows_v, sem).wait()
            pltpu.sync_copy(rows_v, out_hbm.at[pl.ds(base, CH)])
            return carry

        lax.fori_loop(0, nch, body, 0)

    return gk


def _plan(M, D, budget=6 * 1024 * 1024):
    lim_sc = _CHUNK_BYTES // (D * 4)
    ch_star = max(8, lim_sc // 8 * 8)
    best = None
    for ch in range(8, ch_star + 8, 8):
        mpad = _rup(M, ch)
        key = (mpad, -ch)
        if best is None or key < best[0]:
            best = (key, ch, mpad)
    _, CH, Mpad = best
    row_b = _H * _rup(D, 128) * 4
    bm_max = max(8, budget // row_b)
    BM = 8
    for b in range(8, CH + 8, 8):
        if CH % b == 0 and b <= bm_max:
            BM = b
    return CH, Mpad, BM


def _sc_gather(table, idx, CH, Mpad):
    R, D = table.shape
    E = idx.shape[0]
    B = Mpad * _H
    nch = (B // _NW) // CH
    idx_p = jnp.concatenate([idx, jnp.zeros((B - E,), jnp.int32)]) if B > E else idx
    return _gather_kernel(R, D, nch, CH)(table, idx_p)


_DOT = dict(preferred_element_type=jnp.float32,
            precision=lax.Precision.HIGHEST)


def _lrelu(x):
    return jnp.where(x >= 0, x, 0.1 * x)


def _gn(y, gamma, beta):
    n, C = y.shape
    cg = C // _G
    ci = lax.broadcasted_iota(jnp.int32, (C, _G), 0)
    gi = lax.broadcasted_iota(jnp.int32, (C, _G), 1)
    A = jnp.where(ci // cg == gi, 1.0, 0.0).astype(jnp.float32)
    cnt = float(cg * n)
    s1 = jnp.sum(y, axis=0, keepdims=True)
    mean = jnp.dot(jnp.dot(s1, A, **_DOT), A.T, **_DOT) / cnt
    z = y - mean
    s2 = jnp.sum(z * z, axis=0, keepdims=True)
    var = jnp.dot(jnp.dot(s2, A, **_DOT), A.T, **_DOT) / cnt
    return z / jnp.sqrt(var + _EPS) * gamma + beta


def _gn_act_call(x, gamma, beta):
    n, C = x.shape

    def body(x_r, g_r, be_r, o_r):
        o_r[...] = _lrelu(_gn(x_r[...], g_r[...], be_r[...]))

    return pl.pallas_call(
        body, out_shape=jax.ShapeDtypeStruct((n, C), jnp.float32),
    )(x, gamma.reshape(1, C), beta.reshape(1, C))


def _lin_gn_call(x, w, b, gamma, beta, act, shortcut=None):
    n, cin = x.shape
    cout = w.shape[1]
    args = [x, w, b.reshape(1, cout), gamma.reshape(1, cout),
            beta.reshape(1, cout)]
    if shortcut is not None:
        args.append(shortcut)

    def body(*refs):
        if shortcut is not None:
            x_r, w_r, b_r, g_r, be_r, s_r, o_r = refs
        else:
            x_r, w_r, b_r, g_r, be_r, o_r = refs
        y = jnp.dot(x_r[...], w_r[...], **_DOT) + b_r[...]
        y = _gn(y, g_r[...], be_r[...])
        if shortcut is not None:
            y = y + s_r[...]
        if act:
            y = _lrelu(y)
        o_r[...] = y

    return pl.pallas_call(
        body, out_shape=jax.ShapeDtypeStruct((n, cout), jnp.float32),
    )(*args)


def _kpconv_call(g, q, kp, w, sigma, ck, cout, BM):
    E, D = g.shape
    M = E // _H
    grid = M // BM
    wf = w.reshape(_KS * ck, cout)
    inv_sigma = 1.0 / sigma

    def body(g_r, q_r, kp_r, w_r, o_r):
        gv = g_r[...]
        f = gv[:, :ck]
        p = gv[:, ck:ck + 3]
        qv = q_r[...]
        qe = jnp.broadcast_to(qv[:, None, :], (BM, _H, 3)).reshape(BM * _H, 3)
        d = p - qe
        f3 = f.reshape(BM, _H, ck)
        nsum = jnp.sum(f3, axis=2)
        cnt = jnp.sum(jnp.where(nsum > 0.0, 1.0, 0.0), axis=1,
                      keepdims=True)
        cnt = jnp.maximum(cnt, 1.0)
        acc = jnp.zeros((BM, cout), jnp.float32)
        for k in range(_KS):
            dk = d - kp_r[k:k + 1, :]
            sqd = jnp.sum(dk * dk, axis=1, keepdims=True)
            infl_k = jnp.maximum(1.0 - jnp.sqrt(sqd) * inv_sigma, 0.0)
            ek = (f * infl_k).reshape(BM, _H, ck)
            sk = jnp.sum(ek, axis=1)
            acc = acc + jnp.dot(sk, w_r[k * ck:(k + 1) * ck, :], **_DOT)
        o_r[...] = acc / cnt

    return pl.pallas_call(
        body,
        grid=(grid,),
        in_specs=[
            pl.BlockSpec((BM * _H, D), lambda i: (i, 0)),
            pl.BlockSpec((BM, 3), lambda i: (i, 0)),
            pl.BlockSpec((_KS, 3), lambda i: (0, 0)),
            pl.BlockSpec((_KS * ck, cout), lambda i: (0, 0)),
        ],
        out_specs=pl.BlockSpec((BM, cout), lambda i: (i, 0)),
        out_shape=jax.ShapeDtypeStruct((M, cout), jnp.float32),
    )(g, q, kp, wf)


def _maxpool_call(g, cin, BM):
    E, D = g.shape
    M = E // _H
    grid = M // BM

    def body(g_r, o_r):
        x3 = g_r[...].reshape(BM, _H, D)
        o_r[...] = jnp.max(x3, axis=1)

    return pl.pallas_call(
        body,
        grid=(grid,),
        in_specs=[pl.BlockSpec((BM * _H, D), lambda i: (i, 0))],
        out_specs=pl.BlockSpec((BM, D), lambda i: (i, 0)),
        out_shape=jax.ShapeDtypeStruct((M, D), jnp.float32),
    )(g)


def _kp_gather(xfeats, ppad, nbr):
    N1 = ppad.shape[0]
    M = nbr.shape[0]
    ck = xfeats.shape[1]
    D = _rup(ck + 3, 16)
    CH, Mpad, BM = _plan(M, D)
    fpad = jnp.concatenate([xfeats, jnp.zeros((1, ck), jnp.float32)], axis=0)
    cols = [fpad, ppad]
    if D > ck + 3:
        cols.append(jnp.zeros((N1, D - ck - 3), jnp.float32))
    table = jnp.concatenate(cols, axis=1)
    return _sc_gather(table, nbr.reshape(-1), CH, Mpad), BM, Mpad


def _feat_gather(xfeats, nbr):
    cin = xfeats.shape[1]
    M = nbr.shape[0]
    CH, Mpad, BM = _plan(M, cin)
    table = jnp.concatenate(
        [xfeats, jnp.zeros((1, cin), jnp.float32)], axis=0)
    return _sc_gather(table, nbr.reshape(-1), CH, Mpad), BM, Mpad


def _pad_rows(x, Mpad):
    M = x.shape[0]
    if Mpad == M:
        return x
    return jnp.concatenate(
        [x, jnp.zeros((Mpad - M,) + x.shape[1:], x.dtype)], axis=0)


def kernel(feats, points0, points1, points2, points3, points4,
           neighbors0, neighbors1, neighbors2, neighbors3, neighbors4,
           subsampling0, subsampling1, subsampling2, subsampling3, params):
    pts = [points0, points1, points2, points3, points4]
    nbrs = [neighbors0, neighbors1, neighbors2, neighbors3, neighbors4]
    subs = [subsampling0, subsampling1, subsampling2, subsampling3]
    ppad = [jnp.concatenate([p, jnp.full((1, 3), 1000000.0, jnp.float32)],
                            axis=0) for p in pts]

    x = feats
    outs = {}
    for name, btype, cin, cout, rm, ql, sl, nk, ni in _BLOCKS:
        nbr = nbrs[ni] if nk == 'n' else subs[ni]
        sigma = 2.0 * rm
        p = params[name]
        M = nbr.shape[0]
        if btype == 'conv':
            ck = 1
            g, BM, Mpad = _kp_gather(x, ppad[sl], nbr)
            y = _kpconv_call(g, _pad_rows(pts[ql], Mpad), p['kp'], p['w'],
                             sigma, ck, cout, BM)[:M]
            x = _gn_act_call(y, p['g'], p['b'])
        else:
            mid = cout // 4
            inp = x
            u = _lin_gn_call(inp, p['u1_w'], p['u1_b'], p['u1_g'], p['u1_be'],
                             act=True)
            g, BM, Mpad = _kp_gather(u, ppad[sl], nbr)
            y = _kpconv_call(g, _pad_rows(pts[ql], Mpad), p['kp'], p['w'],
                             sigma, mid, mid, BM)[:M]
            y = _gn_act_call(y, p['cg'], p['cb'])
            if nk == 's':
                sg, sBM, sMpad = _feat_gather(inp, nbr)
                sc = _maxpool_call(sg, cin, sBM)[:M]
            else:
                sc = inp
            if 'sc_w' in p:
                sc = _lin_gn_call(sc, p['sc_w'], p['sc_b'], p['sc_g'],
                                  p['sc_be'], act=False)
            x = _lin_gn_call(y, p['u2_w'], p['u2_b'], p['u2_g'], p['u2_be'],
                             act=True, shortcut=sc)
        outs[name] = x
    return (outs['e2_3'], outs['e3_3'], outs['e4_3'], outs['e5_3'])


# speedup vs baseline: 1.7148x; 1.7148x over previous
import functools

import jax
import jax.numpy as jnp
from jax import lax
from jax.experimental import pallas as pl
from jax.experimental.pallas import tpu as pltpu
from jax.experimental.pallas import tpu_sc as plsc

_LEVELS = [10000, 2500, 640, 160, 64]
_H = 32
_KS = 15
_G = 32
_EPS = 1e-5
_NW = 32
_CHUNK_BYTES = 160 * 1024

_BLOCKS = [
    ('e1_1', 'conv', 1, 64, 1, 0, 0, 'n', 0),
    ('e1_2', 'res', 64, 128, 1, 0, 0, 'n', 0),
    ('e2_1', 'res', 128, 128, 1, 1, 0, 's', 0),
    ('e2_2', 'res', 128, 256, 2, 1, 1, 'n', 1),
    ('e2_3', 'res', 256, 256, 2, 1, 1, 'n', 1),
    ('e3_1', 'res', 256, 256, 2, 2, 1, 's', 1),
    ('e3_2', 'res', 256, 512, 4, 2, 2, 'n', 2),
    ('e3_3', 'res', 512, 512, 4, 2, 2, 'n', 2),
    ('e4_1', 'res', 512, 512, 4, 3, 2, 's', 2),
    ('e4_2', 'res', 512, 1024, 8, 3, 3, 'n', 3),
    ('e4_3', 'res', 1024, 1024, 8, 3, 3, 'n', 3),
    ('e5_1', 'res', 1024, 1024, 8, 4, 3, 's', 3),
    ('e5_2', 'res', 1024, 2048, 16, 4, 4, 'n', 4),
    ('e5_3', 'res', 2048, 2048, 16, 4, 4, 'n', 4),
]


def _rup(x, m):
    return (x + m - 1) // m * m


@functools.lru_cache(maxsize=None)
def _gather_kernel(R, D, nch, CH):
    B = _NW * nch * CH
    mesh = plsc.VectorSubcoreMesh(core_axis_name="c", subcore_axis_name="s")

    @functools.partial(
        pl.kernel,
        out_type=jax.ShapeDtypeStruct((B, D), jnp.float32),
        mesh=mesh,
        scratch_types=[
            pltpu.VMEM((CH,), jnp.int32),
            pltpu.VMEM((CH, D), jnp.float32),
            pltpu.SemaphoreType.DMA,
        ],
        compiler_params=pltpu.CompilerParams(use_tc_tiling_on_sc=False),
    )
    def gk(table_hbm, idx_hbm, out_hbm, idx_v, rows_v, sem):
        wid = lax.axis_index("s") * 2 + lax.axis_index("c")

        def body(i, carry):
            base = (wid * nch + i) * CH
            pltpu.sync_copy(idx_hbm.at[pl.ds(base, CH)], idx_v)
            pltpu.async_copy(table_hbm.at[idx_v], rows_v, sem).wait()
            pltpu.sync_copy(rows_v, out_hbm.at[pl.ds(base, CH)])
            return carry

        lax.fori_loop(0, nch, body, 0)

    return gk


def _plan(M, D, budget=6 * 1024 * 1024):
    lim_sc = _CHUNK_BYTES // (D * 4)
    ch_star = max(8, lim_sc // 8 * 8)
    best = None
    for ch in range(8, ch_star + 8, 8):
        mpad = _rup(M, ch)
        key = (mpad, -ch)
        if best is None or key < best[0]:
            best = (key, ch, mpad)
    _, CH, Mpad = best
    row_b = _H * _rup(D, 128) * 4
    bm_max = max(8, budget // row_b)
    BM = 8
    for b in range(8, CH + 8, 8):
        if CH % b == 0 and b <= bm_max:
            BM = b
    return CH, Mpad, BM


def _sc_gather(table, idx, CH, Mpad):
    R, D = table.shape
    E = idx.shape[0]
    B = Mpad * _H
    nch = (B // _NW) // CH
    idx_p = jnp.concatenate([idx, jnp.zeros((B - E,), jnp.int32)]) if B > E else idx
    return _gather_kernel(R, D, nch, CH)(table, idx_p)


_DOT = dict(preferred_element_type=jnp.float32,
            precision=lax.Precision.HIGHEST)


def _lrelu(x):
    return jnp.where(x >= 0, x, 0.1 * x)


def _gn(y, gamma, beta):
    n, C = y.shape
    cg = C // _G
    ci = lax.broadcasted_iota(jnp.int32, (C, _G), 0)
    gi = lax.broadcasted_iota(jnp.int32, (C, _G), 1)
    A = jnp.where(ci // cg == gi, 1.0, 0.0).astype(jnp.float32)
    cnt = float(cg * n)
    s1 = jnp.sum(y, axis=0, keepdims=True)
    mean = jnp.dot(jnp.dot(s1, A, **_DOT), A.T, **_DOT) / cnt
    z = y - mean
    s2 = jnp.sum(z * z, axis=0, keepdims=True)
    var = jnp.dot(jnp.dot(s2, A, **_DOT), A.T, **_DOT) / cnt
    return z / jnp.sqrt(var + _EPS) * gamma + beta


def _gn_act_call(x, gamma, beta):
    n, C = x.shape

    def body(x_r, g_r, be_r, o_r):
        o_r[...] = _lrelu(_gn(x_r[...], g_r[...], be_r[...]))

    return pl.pallas_call(
        body, out_shape=jax.ShapeDtypeStruct((n, C), jnp.float32),
    )(x, gamma.reshape(1, C), beta.reshape(1, C))


def _lin_gn_call(x, w, b, gamma, beta, act, shortcut=None):
    n, cin = x.shape
    cout = w.shape[1]
    args = [x, w, b.reshape(1, cout), gamma.reshape(1, cout),
            beta.reshape(1, cout)]
    if shortcut is not None:
        args.append(shortcut)

    def body(*refs):
        if shortcut is not None:
            x_r, w_r, b_r, g_r, be_r, s_r, o_r = refs
        else:
            x_r, w_r, b_r, g_r, be_r, o_r = refs
        y = jnp.dot(x_r[...], w_r[...], **_DOT) + b_r[...]
        y = _gn(y, g_r[...], be_r[...])
        if shortcut is not None:
            y = y + s_r[...]
        if act:
            y = _lrelu(y)
        o_r[...] = y

    return pl.pallas_call(
        body, out_shape=jax.ShapeDtypeStruct((n, cout), jnp.float32),
    )(*args)


def _kpconv_call(g, q, kp, w, sigma, ck, cout, BM):
    E, D = g.shape
    M = E // _H
    grid = M // BM
    wf = w.reshape(_KS * ck, cout)
    kpt = kp.T
    kn = jnp.sum(kp * kp, axis=1).reshape(1, _KS)
    inv_sigma = 1.0 / sigma

    def body(g_r, q_r, kpt_r, kn_r, w_r, o_r):
        gv = g_r[...]
        f = gv[:, :ck]
        p = gv[:, ck:ck + 3]
        qv = q_r[...]
        qe = jnp.broadcast_to(qv[:, None, :], (BM, _H, 3)).reshape(BM * _H, 3)
        d = p - qe
        p2 = jnp.sum(d * d, axis=1, keepdims=True)
        dkp = jnp.dot(d, kpt_r[...], **_DOT)
        sqd = jnp.maximum(p2 - 2.0 * dkp + kn_r[...], 0.0)
        infl = jnp.maximum(1.0 - jnp.sqrt(sqd) * inv_sigma, 0.0)
        i3 = infl.reshape(BM, _H, _KS)
        f3 = f.reshape(BM, _H, ck)
        weighted = lax.dot_general(
            i3, f3, (((1,), (1,)), ((0,), (0,))),
            precision=lax.Precision.HIGHEST,
            preferred_element_type=jnp.float32)
        nsum = jnp.sum(f3, axis=2)
        cnt = jnp.sum(jnp.where(nsum > 0.0, 1.0, 0.0), axis=1,
                      keepdims=True)
        cnt = jnp.maximum(cnt, 1.0)
        acc = jnp.zeros((BM, cout), jnp.float32)
        for k in range(_KS):
            acc = acc + jnp.dot(weighted[:, k, :],
                                w_r[k * ck:(k + 1) * ck, :], **_DOT)
        o_r[...] = acc / cnt

    return pl.pallas_call(
        body,
        grid=(grid,),
        in_specs=[
            pl.BlockSpec((BM * _H, D), lambda i: (i, 0)),
            pl.BlockSpec((BM, 3), lambda i: (i, 0)),
            pl.BlockSpec((3, _KS), lambda i: (0, 0)),
            pl.BlockSpec((1, _KS), lambda i: (0, 0)),
            pl.BlockSpec((_KS * ck, cout), lambda i: (0, 0)),
        ],
        out_specs=pl.BlockSpec((BM, cout), lambda i: (i, 0)),
        out_shape=jax.ShapeDtypeStruct((M, cout), jnp.float32),
    )(g, q, kpt, kn, wf)


def _maxpool_call(g, cin, BM):
    E, D = g.shape
    M = E // _H
    grid = M // BM

    def body(g_r, o_r):
        x3 = g_r[...].reshape(BM, _H, D)
        o_r[...] = jnp.max(x3, axis=1)

    return pl.pallas_call(
        body,
        grid=(grid,),
        in_specs=[pl.BlockSpec((BM * _H, D), lambda i: (i, 0))],
        out_specs=pl.BlockSpec((BM, D), lambda i: (i, 0)),
        out_shape=jax.ShapeDtypeStruct((M, D), jnp.float32),
    )(g)


def _kp_gather(xfeats, ppad, nbr):
    N1 = ppad.shape[0]
    M = nbr.shape[0]
    ck = xfeats.shape[1]
    D = _rup(ck + 3, 16)
    CH, Mpad, BM = _plan(M, D)
    fpad = jnp.concatenate([xfeats, jnp.zeros((1, ck), jnp.float32)], axis=0)
    cols = [fpad, ppad]
    if D > ck + 3:
        cols.append(jnp.zeros((N1, D - ck - 3), jnp.float32))
    table = jnp.concatenate(cols, axis=1)
    return _sc_gather(table, nbr.reshape(-1), CH, Mpad), BM, Mpad


def _feat_gather(xfeats, nbr):
    cin = xfeats.shape[1]
    M = nbr.shape[0]
    CH, Mpad, BM = _plan(M, cin)
    table = jnp.concatenate(
        [xfeats, jnp.zeros((1, cin), jnp.float32)], axis=0)
    return _sc_gather(table, nbr.reshape(-1), CH, Mpad), BM, Mpad


def _pad_rows(x, Mpad):
    M = x.shape[0]
    if Mpad == M:
        return x
    return jnp.concatenate(
        [x, jnp.zeros((Mpad - M,) + x.shape[1:], x.dtype)], axis=0)


def kernel(feats, points0, points1, points2, points3, points4,
           neighbors0, neighbors1, neighbors2, neighbors3, neighbors4,
           subsampling0, subsampling1, subsampling2, subsampling3, params):
    pts = [points0, points1, points2, points3, points4]
    nbrs = [neighbors0, neighbors1, neighbors2, neighbors3, neighbors4]
    subs = [subsampling0, subsampling1, subsampling2, subsampling3]
    ppad = [jnp.concatenate([p, jnp.full((1, 3), 1000000.0, jnp.float32)],
                            axis=0) for p in pts]

    x = feats
    outs = {}
    for name, btype, cin, cout, rm, ql, sl, nk, ni in _BLOCKS:
        nbr = nbrs[ni] if nk == 'n' else subs[ni]
        sigma = 2.0 * rm
        p = params[name]
        M = nbr.shape[0]
        if btype == 'conv':
            ck = 1
            g, BM, Mpad = _kp_gather(x, ppad[sl], nbr)
            y = _kpconv_call(g, _pad_rows(pts[ql], Mpad), p['kp'], p['w'],
                             sigma, ck, cout, BM)[:M]
            x = _gn_act_call(y, p['g'], p['b'])
        else:
            mid = cout // 4
            inp = x
            u = _lin_gn_call(inp, p['u1_w'], p['u1_b'], p['u1_g'], p['u1_be'],
                             act=True)
            g, BM, Mpad = _kp_gather(u, ppad[sl], nbr)
            y = _kpconv_call(g, _pad_rows(pts[ql], Mpad), p['kp'], p['w'],
                             sigma, mid, mid, BM)[:M]
            y = _gn_act_call(y, p['cg'], p['cb'])
            if nk == 's':
                sg, sBM, sMpad = _feat_gather(inp, nbr)
                sc = _maxpool_call(sg, cin, sBM)[:M]
            else:
                sc = inp
            if 'sc_w' in p:
                sc = _lin_gn_call(sc, p['sc_w'], p['sc_b'], p['sc_g'],
                                  p['sc_be'], act=False)
            x = _lin_gn_call(y, p['u2_w'], p['u2_b'], p['u2_g'], p['u2_be'],
                             act=True, shortcut=sc)
        outs[name] = x
    return (outs['e2_3'], outs['e3_3'], outs['e4_3'], outs['e5_3'])


# speedup vs baseline: 1.9938x; 1.1627x over previous
import functools

import jax
import jax.numpy as jnp
from jax import lax
from jax.experimental import pallas as pl
from jax.experimental.pallas import tpu as pltpu
from jax.experimental.pallas import tpu_sc as plsc

_LEVELS = [10000, 2500, 640, 160, 64]
_H = 32
_KS = 15
_G = 32
_EPS = 1e-5
_NW = 32
_CHUNK_BYTES = 160 * 1024

_BLOCKS = [
    ('e1_1', 'conv', 1, 64, 1, 0, 0, 'n', 0),
    ('e1_2', 'res', 64, 128, 1, 0, 0, 'n', 0),
    ('e2_1', 'res', 128, 128, 1, 1, 0, 's', 0),
    ('e2_2', 'res', 128, 256, 2, 1, 1, 'n', 1),
    ('e2_3', 'res', 256, 256, 2, 1, 1, 'n', 1),
    ('e3_1', 'res', 256, 256, 2, 2, 1, 's', 1),
    ('e3_2', 'res', 256, 512, 4, 2, 2, 'n', 2),
    ('e3_3', 'res', 512, 512, 4, 2, 2, 'n', 2),
    ('e4_1', 'res', 512, 512, 4, 3, 2, 's', 2),
    ('e4_2', 'res', 512, 1024, 8, 3, 3, 'n', 3),
    ('e4_3', 'res', 1024, 1024, 8, 3, 3, 'n', 3),
    ('e5_1', 'res', 1024, 1024, 8, 4, 3, 's', 3),
    ('e5_2', 'res', 1024, 2048, 16, 4, 4, 'n', 4),
    ('e5_3', 'res', 2048, 2048, 16, 4, 4, 'n', 4),
]


def _rup(x, m):
    return (x + m - 1) // m * m


@functools.lru_cache(maxsize=None)
def _gather_kernel(R, D, nch, CH):
    B = _NW * nch * CH
    mesh = plsc.VectorSubcoreMesh(core_axis_name="c", subcore_axis_name="s")

    @functools.partial(
        pl.kernel,
        out_type=jax.ShapeDtypeStruct((B, D), jnp.float32),
        mesh=mesh,
        scratch_types=[
            pltpu.VMEM((nch * CH,), jnp.int32),
            pltpu.VMEM((CH, D), jnp.float32),
            pltpu.VMEM((CH, D), jnp.float32),
            pltpu.SemaphoreType.DMA,
            pltpu.SemaphoreType.DMA,
            pltpu.SemaphoreType.DMA,
            pltpu.SemaphoreType.DMA,
        ],
        compiler_params=pltpu.CompilerParams(use_tc_tiling_on_sc=False),
    )
    def gk(table_hbm, idx_hbm, out_hbm, idx_v, rows0, rows1,
           g0, g1, w0, w1):
        wid = lax.axis_index("s") * 2 + lax.axis_index("c")
        base_w = wid * (nch * CH)
        pltpu.sync_copy(idx_hbm.at[pl.ds(base_w, nch * CH)], idx_v)
        rows = (rows0, rows1)
        gsem = (g0, g1)
        wsem = (w0, w1)
        gd, wd = {}, {}

        def start(i):
            b = i % 2
            gd[i] = pltpu.async_copy(
                table_hbm.at[idx_v.at[pl.ds(i * CH, CH)]], rows[b], gsem[b])

        start(0)
        for i in range(nch):
            b = i % 2
            if i + 1 < nch:
                if i - 1 >= 0:
                    wd[i - 1].wait()
                start(i + 1)
            gd[i].wait()
            wd[i] = pltpu.async_copy(
                rows[b], out_hbm.at[pl.ds(base_w + i * CH, CH)], wsem[b])
        wd[nch - 1].wait()
        if nch >= 2:
            wd[nch - 2].wait()

    return gk


def _plan(M, D, budget=6 * 1024 * 1024):
    lim_sc = _CHUNK_BYTES // (D * 4)
    ch_star = max(8, lim_sc // 8 * 8)
    best = None
    for ch in range(8, ch_star + 8, 8):
        mpad = _rup(M, ch)
        key = (mpad, -ch)
        if best is None or key < best[0]:
            best = (key, ch, mpad)
    _, CH, Mpad = best
    row_b = _H * _rup(D, 128) * 4
    bm_max = max(8, budget // row_b)
    BM = 8
    for b in range(8, CH + 8, 8):
        if CH % b == 0 and b <= bm_max:
            BM = b
    return CH, Mpad, BM


def _sc_gather(table, idx, CH, Mpad):
    R, D = table.shape
    E = idx.shape[0]
    B = Mpad * _H
    nch = (B // _NW) // CH
    idx_p = jnp.concatenate([idx, jnp.zeros((B - E,), jnp.int32)]) if B > E else idx
    return _gather_kernel(R, D, nch, CH)(table, idx_p)


_DOT = dict(preferred_element_type=jnp.float32,
            precision=lax.Precision.HIGHEST)


def _lrelu(x):
    return jnp.where(x >= 0, x, 0.1 * x)


def _gn(y, gamma, beta):
    n, C = y.shape
    cg = C // _G
    ci = lax.broadcasted_iota(jnp.int32, (C, _G), 0)
    gi = lax.broadcasted_iota(jnp.int32, (C, _G), 1)
    A = jnp.where(ci // cg == gi, 1.0, 0.0).astype(jnp.float32)
    cnt = float(cg * n)
    s1 = jnp.sum(y, axis=0, keepdims=True)
    mean = jnp.dot(jnp.dot(s1, A, **_DOT), A.T, **_DOT) / cnt
    z = y - mean
    s2 = jnp.sum(z * z, axis=0, keepdims=True)
    var = jnp.dot(jnp.dot(s2, A, **_DOT), A.T, **_DOT) / cnt
    return z / jnp.sqrt(var + _EPS) * gamma + beta


def _gn_act_call(x, gamma, beta):
    n, C = x.shape

    def body(x_r, g_r, be_r, o_r):
        o_r[...] = _lrelu(_gn(x_r[...], g_r[...], be_r[...]))

    return pl.pallas_call(
        body, out_shape=jax.ShapeDtypeStruct((n, C), jnp.float32),
    )(x, gamma.reshape(1, C), beta.reshape(1, C))


def _lin_gn_call(x, w, b, gamma, beta, act, shortcut=None):
    n, cin = x.shape
    cout = w.shape[1]
    args = [x, w, b.reshape(1, cout), gamma.reshape(1, cout),
            beta.reshape(1, cout)]
    if shortcut is not None:
        args.append(shortcut)

    def body(*refs):
        if shortcut is not None:
            x_r, w_r, b_r, g_r, be_r, s_r, o_r = refs
        else:
            x_r, w_r, b_r, g_r, be_r, o_r = refs
        y = jnp.dot(x_r[...], w_r[...], **_DOT) + b_r[...]
        y = _gn(y, g_r[...], be_r[...])
        if shortcut is not None:
            y = y + s_r[...]
        if act:
            y = _lrelu(y)
        o_r[...] = y

    return pl.pallas_call(
        body, out_shape=jax.ShapeDtypeStruct((n, cout), jnp.float32),
    )(*args)


def _kpconv_call(g, q, kp, w, sigma, ck, cout, BM):
    E, D = g.shape
    M = E // _H
    grid = M // BM
    wf = w.reshape(_KS * ck, cout)
    kpt = kp.T
    kn = jnp.sum(kp * kp, axis=1).reshape(1, _KS)
    inv_sigma = 1.0 / sigma

    def body(g_r, q_r, kpt_r, kn_r, w_r, o_r):
        gv = g_r[...]
        f = gv[:, :ck]
        p = gv[:, ck:ck + 3]
        qv = q_r[...]
        qe = jnp.broadcast_to(qv[:, None, :], (BM, _H, 3)).reshape(BM * _H, 3)
        d = p - qe
        p2 = jnp.sum(d * d, axis=1, keepdims=True)
        dkp = jnp.dot(d, kpt_r[...], **_DOT)
        sqd = jnp.maximum(p2 - 2.0 * dkp + kn_r[...], 0.0)
        infl = jnp.maximum(1.0 - jnp.sqrt(sqd) * inv_sigma, 0.0)
        i3 = infl.reshape(BM, _H, _KS)
        f3 = f.reshape(BM, _H, ck)
        weighted = lax.dot_general(
            i3, f3, (((1,), (1,)), ((0,), (0,))),
            precision=lax.Precision.HIGHEST,
            preferred_element_type=jnp.float32)
        nsum = jnp.sum(f3, axis=2)
        cnt = jnp.sum(jnp.where(nsum > 0.0, 1.0, 0.0), axis=1,
                      keepdims=True)
        cnt = jnp.maximum(cnt, 1.0)
        acc = jnp.zeros((BM, cout), jnp.float32)
        for k in range(_KS):
            acc = acc + jnp.dot(weighted[:, k, :],
                                w_r[k * ck:(k + 1) * ck, :], **_DOT)
        o_r[...] = acc / cnt

    return pl.pallas_call(
        body,
        grid=(grid,),
        in_specs=[
            pl.BlockSpec((BM * _H, D), lambda i: (i, 0)),
            pl.BlockSpec((BM, 3), lambda i: (i, 0)),
            pl.BlockSpec((3, _KS), lambda i: (0, 0)),
            pl.BlockSpec((1, _KS), lambda i: (0, 0)),
            pl.BlockSpec((_KS * ck, cout), lambda i: (0, 0)),
        ],
        out_specs=pl.BlockSpec((BM, cout), lambda i: (i, 0)),
        out_shape=jax.ShapeDtypeStruct((M, cout), jnp.float32),
    )(g, q, kpt, kn, wf)


def _maxpool_call(g, cin, BM):
    E, D = g.shape
    M = E // _H
    grid = M // BM

    def body(g_r, o_r):
        x3 = g_r[...].reshape(BM, _H, D)
        o_r[...] = jnp.max(x3, axis=1)

    return pl.pallas_call(
        body,
        grid=(grid,),
        in_specs=[pl.BlockSpec((BM * _H, D), lambda i: (i, 0))],
        out_specs=pl.BlockSpec((BM, D), lambda i: (i, 0)),
        out_shape=jax.ShapeDtypeStruct((M, D), jnp.float32),
    )(g)


def _kp_gather(xfeats, ppad, nbr):
    N1 = ppad.shape[0]
    M = nbr.shape[0]
    ck = xfeats.shape[1]
    D = _rup(ck + 3, 16)
    CH, Mpad, BM = _plan(M, D)
    fpad = jnp.concatenate([xfeats, jnp.zeros((1, ck), jnp.float32)], axis=0)
    cols = [fpad, ppad]
    if D > ck + 3:
        cols.append(jnp.zeros((N1, D - ck - 3), jnp.float32))
    table = jnp.concatenate(cols, axis=1)
    return _sc_gather(table, nbr.reshape(-1), CH, Mpad), BM, Mpad


def _feat_gather(xfeats, nbr):
    cin = xfeats.shape[1]
    M = nbr.shape[0]
    CH, Mpad, BM = _plan(M, cin)
    table = jnp.concatenate(
        [xfeats, jnp.zeros((1, cin), jnp.float32)], axis=0)
    return _sc_gather(table, nbr.reshape(-1), CH, Mpad), BM, Mpad


def _pad_rows(x, Mpad):
    M = x.shape[0]
    if Mpad == M:
        return x
    return jnp.concatenate(
        [x, jnp.zeros((Mpad - M,) + x.shape[1:], x.dtype)], axis=0)


def kernel(feats, points0, points1, points2, points3, points4,
           neighbors0, neighbors1, neighbors2, neighbors3, neighbors4,
           subsampling0, subsampling1, subsampling2, subsampling3, params):
    pts = [points0, points1, points2, points3, points4]
    nbrs = [neighbors0, neighbors1, neighbors2, neighbors3, neighbors4]
    subs = [subsampling0, subsampling1, subsampling2, subsampling3]
    ppad = [jnp.concatenate([p, jnp.full((1, 3), 1000000.0, jnp.float32)],
                            axis=0) for p in pts]

    x = feats
    outs = {}
    for name, btype, cin, cout, rm, ql, sl, nk, ni in _BLOCKS:
        nbr = nbrs[ni] if nk == 'n' else subs[ni]
        sigma = 2.0 * rm
        p = params[name]
        M = nbr.shape[0]
        if btype == 'conv':
            ck = 1
            g, BM, Mpad = _kp_gather(x, ppad[sl], nbr)
            y = _kpconv_call(g, _pad_rows(pts[ql], Mpad), p['kp'], p['w'],
                             sigma, ck, cout, BM)[:M]
            x = _gn_act_call(y, p['g'], p['b'])
        else:
            mid = cout // 4
            inp = x
            u = _lin_gn_call(inp, p['u1_w'], p['u1_b'], p['u1_g'], p['u1_be'],
                             act=True)
            g, BM, Mpad = _kp_gather(u, ppad[sl], nbr)
            y = _kpconv_call(g, _pad_rows(pts[ql], Mpad), p['kp'], p['w'],
                             sigma, mid, mid, BM)[:M]
            y = _gn_act_call(y, p['cg'], p['cb'])
            if nk == 's':
                sg, sBM, sMpad = _feat_gather(inp, nbr)
                sc = _maxpool_call(sg, cin, sBM)[:M]
            else:
                sc = inp
            if 'sc_w' in p:
                sc = _lin_gn_call(sc, p['sc_w'], p['sc_b'], p['sc_g'],
                                  p['sc_be'], act=False)
            x = _lin_gn_call(y, p['u2_w'], p['u2_b'], p['u2_g'], p['u2_be'],
                             act=True, shortcut=sc)
        outs[name] = x
    return (outs['e2_3'], outs['e3_3'], outs['e4_3'], outs['e5_3'])


# speedup vs baseline: 3.5165x; 1.7637x over previous
import functools

import jax
import jax.numpy as jnp
from jax import lax
from jax.experimental import pallas as pl
from jax.experimental.pallas import tpu as pltpu
from jax.experimental.pallas import tpu_sc as plsc

_LEVELS = [10000, 2500, 640, 160, 64]
_H = 32
_KS = 15
_G = 32
_EPS = 1e-5
_NW = 32
_CHUNK_BYTES = 192 * 1024

_BLOCKS = [
    ('e1_1', 'conv', 1, 64, 1, 0, 0, 'n', 0),
    ('e1_2', 'res', 64, 128, 1, 0, 0, 'n', 0),
    ('e2_1', 'res', 128, 128, 1, 1, 0, 's', 0),
    ('e2_2', 'res', 128, 256, 2, 1, 1, 'n', 1),
    ('e2_3', 'res', 256, 256, 2, 1, 1, 'n', 1),
    ('e3_1', 'res', 256, 256, 2, 2, 1, 's', 1),
    ('e3_2', 'res', 256, 512, 4, 2, 2, 'n', 2),
    ('e3_3', 'res', 512, 512, 4, 2, 2, 'n', 2),
    ('e4_1', 'res', 512, 512, 4, 3, 2, 's', 2),
    ('e4_2', 'res', 512, 1024, 8, 3, 3, 'n', 3),
    ('e4_3', 'res', 1024, 1024, 8, 3, 3, 'n', 3),
    ('e5_1', 'res', 1024, 1024, 8, 4, 3, 's', 3),
    ('e5_2', 'res', 1024, 2048, 16, 4, 4, 'n', 4),
    ('e5_3', 'res', 2048, 2048, 16, 4, 4, 'n', 4),
]


def _rup(x, m):
    return (x + m - 1) // m * m


@functools.lru_cache(maxsize=None)
def _gather_kernel(R, D, nch, CH):
    B = _NW * nch * CH
    mesh = plsc.VectorSubcoreMesh(core_axis_name="c", subcore_axis_name="s")

    @functools.partial(
        pl.kernel,
        out_type=jax.ShapeDtypeStruct((B, D), jnp.float32),
        mesh=mesh,
        scratch_types=[
            pltpu.VMEM((nch * CH,), jnp.int32),
            pltpu.VMEM((CH, D), jnp.float32),
            pltpu.VMEM((CH, D), jnp.float32),
            pltpu.SemaphoreType.DMA,
            pltpu.SemaphoreType.DMA,
            pltpu.SemaphoreType.DMA,
            pltpu.SemaphoreType.DMA,
        ],
        compiler_params=pltpu.CompilerParams(use_tc_tiling_on_sc=False),
    )
    def gk(table_hbm, idx_hbm, out_hbm, idx_v, rows0, rows1,
           g0, g1, w0, w1):
        wid = lax.axis_index("s") * 2 + lax.axis_index("c")
        base_w = wid * (nch * CH)
        pltpu.sync_copy(idx_hbm.at[pl.ds(base_w, nch * CH)], idx_v)
        rows = (rows0, rows1)
        gsem = (g0, g1)
        wsem = (w0, w1)
        gd, wd = {}, {}

        def start(i):
            b = i % 2
            gd[i] = pltpu.async_copy(
                table_hbm.at[idx_v.at[pl.ds(i * CH, CH)]], rows[b], gsem[b])

        start(0)
        for i in range(nch):
            b = i % 2
            if i + 1 < nch:
                if i - 1 >= 0:
                    wd[i - 1].wait()
                start(i + 1)
            gd[i].wait()
            wd[i] = pltpu.async_copy(
                rows[b], out_hbm.at[pl.ds(base_w + i * CH, CH)], wsem[b])
        wd[nch - 1].wait()
        if nch >= 2:
            wd[nch - 2].wait()

    return gk


def _plan(M, D, budget=12 * 1024 * 1024):
    lim_sc = _CHUNK_BYTES // (D * 4)
    ch_star = max(8, lim_sc // 8 * 8)
    best = None
    for ch in range(8, ch_star + 8, 8):
        mpad = _rup(M, ch)
        key = (mpad, -ch)
        if best is None or key < best[0]:
            best = (key, ch, mpad)
    _, CH, Mpad = best
    row_b = _H * _rup(D, 128) * 4
    bm_max = max(8, budget // row_b)
    BM = 8
    for b in range(8, CH + 8, 8):
        if CH % b == 0 and b <= bm_max:
            BM = b
    return CH, Mpad, BM


def _sc_gather(table, idx, CH, Mpad):
    R, D = table.shape
    E = idx.shape[0]
    B = Mpad * _H
    nch = (B // _NW) // CH
    idx_p = jnp.concatenate([idx, jnp.zeros((B - E,), jnp.int32)]) if B > E else idx
    return _gather_kernel(R, D, nch, CH)(table, idx_p)


_DOT = dict(preferred_element_type=jnp.float32,
            precision=lax.Precision.HIGHEST)
_DOT3 = dict(preferred_element_type=jnp.float32,
             precision=lax.Precision.DEFAULT)


def _lrelu(x):
    return jnp.where(x >= 0, x, 0.1 * x)


def _gn(y, gamma, beta):
    n, C = y.shape
    cg = C // _G
    ci = lax.broadcasted_iota(jnp.int32, (C, _G), 0)
    gi = lax.broadcasted_iota(jnp.int32, (C, _G), 1)
    A = jnp.where(ci // cg == gi, 1.0, 0.0).astype(jnp.float32)
    cnt = float(cg * n)
    s1 = jnp.sum(y, axis=0, keepdims=True)
    mean = jnp.dot(jnp.dot(s1, A, **_DOT), A.T, **_DOT) / cnt
    z = y - mean
    s2 = jnp.sum(z * z, axis=0, keepdims=True)
    var = jnp.dot(jnp.dot(s2, A, **_DOT), A.T, **_DOT) / cnt
    return z / jnp.sqrt(var + _EPS) * gamma + beta


def _gn_act_call(x, gamma, beta):
    n, C = x.shape

    def body(x_r, g_r, be_r, o_r):
        o_r[...] = _lrelu(_gn(x_r[...], g_r[...], be_r[...]))

    return pl.pallas_call(
        body, out_shape=jax.ShapeDtypeStruct((n, C), jnp.float32),
    )(x, gamma.reshape(1, C), beta.reshape(1, C))


def _lin_gn_call(x, w, b, gamma, beta, act, shortcut=None):
    n, cin = x.shape
    cout = w.shape[1]
    args = [x, w, b.reshape(1, cout), gamma.reshape(1, cout),
            beta.reshape(1, cout)]
    if shortcut is not None:
        args.append(shortcut)

    def body(*refs):
        if shortcut is not None:
            x_r, w_r, b_r, g_r, be_r, s_r, o_r = refs
        else:
            x_r, w_r, b_r, g_r, be_r, o_r = refs
        y = jnp.dot(x_r[...], w_r[...], **_DOT) + b_r[...]
        y = _gn(y, g_r[...], be_r[...])
        if shortcut is not None:
            y = y + s_r[...]
        if act:
            y = _lrelu(y)
        o_r[...] = y

    return pl.pallas_call(
        body, out_shape=jax.ShapeDtypeStruct((n, cout), jnp.float32),
    )(*args)


def _kpconv_call(g, q, kp, w, sigma, ck, cout, BM):
    E, D = g.shape
    M = E // _H
    grid = M // BM
    wf = w.reshape(_KS * ck, cout)
    kpt = kp.T
    kn = jnp.sum(kp * kp, axis=1).reshape(1, _KS)
    inv_sigma = 1.0 / sigma

    def body(g_r, q_r, kpt_r, kn_r, w_r, o_r):
        gv = g_r[...]
        f = gv[:, :ck]
        p = gv[:, ck:ck + 3]
        qv = q_r[...]
        qe = jnp.broadcast_to(qv[:, None, :], (BM, _H, 3)).reshape(BM * _H, 3)
        d = p - qe
        p2 = jnp.sum(d * d, axis=1, keepdims=True)
        dkp = jnp.dot(d, kpt_r[...], **_DOT3)
        sqd = jnp.maximum(p2 - 2.0 * dkp + kn_r[...], 0.0)
        infl = jnp.maximum(1.0 - jnp.sqrt(sqd) * inv_sigma, 0.0)
        i3 = infl.reshape(BM, _H, _KS)
        f3 = f.reshape(BM, _H, ck)
        weighted = lax.dot_general(
            i3, f3, (((1,), (1,)), ((0,), (0,))),
            precision=lax.Precision.DEFAULT,
            preferred_element_type=jnp.float32)
        nsum = jnp.sum(f3, axis=2)
        cnt = jnp.sum(jnp.where(nsum > 0.0, 1.0, 0.0), axis=1,
                      keepdims=True)
        cnt = jnp.maximum(cnt, 1.0)
        acc = jnp.zeros((BM, cout), jnp.float32)
        for k in range(_KS):
            acc = acc + jnp.dot(weighted[:, k, :],
                                w_r[k * ck:(k + 1) * ck, :], **_DOT3)
        o_r[...] = acc / cnt

    return pl.pallas_call(
        body,
        grid=(grid,),
        in_specs=[
            pl.BlockSpec((BM * _H, D), lambda i: (i, 0)),
            pl.BlockSpec((BM, 3), lambda i: (i, 0)),
            pl.BlockSpec((3, _KS), lambda i: (0, 0)),
            pl.BlockSpec((1, _KS), lambda i: (0, 0)),
            pl.BlockSpec((_KS * ck, cout), lambda i: (0, 0)),
        ],
        out_specs=pl.BlockSpec((BM, cout), lambda i: (i, 0)),
        out_shape=jax.ShapeDtypeStruct((M, cout), jnp.float32),
    )(g, q, kpt, kn, wf)


def _maxpool_call(g, cin, BM):
    E, D = g.shape
    M = E // _H
    grid = M // BM

    def body(g_r, o_r):
        x3 = g_r[...].reshape(BM, _H, D)
        o_r[...] = jnp.max(x3, axis=1)

    return pl.pallas_call(
        body,
        grid=(grid,),
        in_specs=[pl.BlockSpec((BM * _H, D), lambda i: (i, 0))],
        out_specs=pl.BlockSpec((BM, D), lambda i: (i, 0)),
        out_shape=jax.ShapeDtypeStruct((M, D), jnp.float32),
    )(g)


def _kp_gather(xfeats, ppad, nbr):
    N1 = ppad.shape[0]
    M = nbr.shape[0]
    ck = xfeats.shape[1]
    D = _rup(ck + 3, 16)
    CH, Mpad, BM = _plan(M, D)
    fpad = jnp.concatenate([xfeats, jnp.zeros((1, ck), jnp.float32)], axis=0)
    cols = [fpad, ppad]
    if D > ck + 3:
        cols.append(jnp.zeros((N1, D - ck - 3), jnp.float32))
    table = jnp.concatenate(cols, axis=1)
    return _sc_gather(table, nbr.reshape(-1), CH, Mpad), BM, Mpad


def _feat_gather(xfeats, nbr):
    cin = xfeats.shape[1]
    M = nbr.shape[0]
    CH, Mpad, BM = _plan(M, cin)
    table = jnp.concatenate(
        [xfeats, jnp.zeros((1, cin), jnp.float32)], axis=0)
    return _sc_gather(table, nbr.reshape(-1), CH, Mpad), BM, Mpad


def _pad_rows(x, Mpad):
    M = x.shape[0]
    if Mpad == M:
        return x
    return jnp.concatenate(
        [x, jnp.zeros((Mpad - M,) + x.shape[1:], x.dtype)], axis=0)


def kernel(feats, points0, points1, points2, points3, points4,
           neighbors0, neighbors1, neighbors2, neighbors3, neighbors4,
           subsampling0, subsampling1, subsampling2, subsampling3, params):
    pts = [points0, points1, points2, points3, points4]
    nbrs = [neighbors0, neighbors1, neighbors2, neighbors3, neighbors4]
    subs = [subsampling0, subsampling1, subsampling2, subsampling3]
    ppad = [jnp.concatenate([p, jnp.full((1, 3), 1000000.0, jnp.float32)],
                            axis=0) for p in pts]

    x = feats
    outs = {}
    for name, btype, cin, cout, rm, ql, sl, nk, ni in _BLOCKS:
        nbr = nbrs[ni] if nk == 'n' else subs[ni]
        sigma = 2.0 * rm
        p = params[name]
        M = nbr.shape[0]
        if btype == 'conv':
            ck = 1
            g, BM, Mpad = _kp_gather(x, ppad[sl], nbr)
            y = _kpconv_call(g, _pad_rows(pts[ql], Mpad), p['kp'], p['w'],
                             sigma, ck, cout, BM)[:M]
            x = _gn_act_call(y, p['g'], p['b'])
        else:
            mid = cout // 4
            inp = x
            u = _lin_gn_call(inp, p['u1_w'], p['u1_b'], p['u1_g'], p['u1_be'],
                             act=True)
            g, BM, Mpad = _kp_gather(u, ppad[sl], nbr)
            y = _kpconv_call(g, _pad_rows(pts[ql], Mpad), p['kp'], p['w'],
                             sigma, mid, mid, BM)[:M]
            y = _gn_act_call(y, p['cg'], p['cb'])
            if nk == 's':
                sg, sBM, sMpad = _feat_gather(inp, nbr)
                sc = _maxpool_call(sg, cin, sBM)[:M]
            else:
                sc = inp
            if 'sc_w' in p:
                sc = _lin_gn_call(sc, p['sc_w'], p['sc_b'], p['sc_g'],
                                  p['sc_be'], act=False)
            x = _lin_gn_call(y, p['u2_w'], p['u2_b'], p['u2_g'], p['u2_be'],
                             act=True, shortcut=sc)
        outs[name] = x
    return (outs['e2_3'], outs['e3_3'], outs['e4_3'], outs['e5_3'])
